# Initial kernel scaffold; baseline (speedup 1.0000x reference)
import jax
import jax.numpy as jnp
from jax.experimental import pallas as pl


def kernel(inputs, targets):
    raise NotImplementedError("write your pallas kernel here")


# speedup vs baseline: 12.4452x; 12.4452x over previous
import functools

import jax
import jax.numpy as jnp
from jax import lax
from jax.experimental import pallas as pl
from jax.experimental.pallas import tpu as pltpu
from jax.experimental.pallas import tpu_sc as plsc

S = 16
P = 512 * 512
HALF = P // 2
NB = 16384
HB = NB // 2
CH = 8192
NCH = HALF // CH
L = 16

_mesh = plsc.VectorSubcoreMesh(core_axis_name="c", subcore_axis_name="s")


@functools.partial(
    pl.kernel,
    out_type=jax.ShapeDtypeStruct((S, L), jnp.float32),
    mesh=_mesh,
    compiler_params=pltpu.CompilerParams(needs_layout_passes=False),
    scratch_types=[
        pltpu.VMEM((CH,), jnp.float32),
        pltpu.VMEM((CH,), jnp.int32),
        pltpu.VMEM((2 * NB,), jnp.int32),
        pltpu.VMEM((NB,), jnp.float32),
        pltpu.VMEM((L,), jnp.float32),
        pltpu.VMEM((L,), jnp.float32),
        pltpu.VMEM_SHARED((16 * 2 * NB,), jnp.int32),
        pltpu.VMEM_SHARED((16 * NB,), jnp.float32),
        pltpu.VMEM_SHARED((16 * L,), jnp.float32),
        pltpu.VMEM_SHARED((16 * L,), jnp.float32),
        pltpu.VMEM_SHARED((16 * L,), jnp.float32),
    ],
)
def _lovasz_sc(x_hbm, t_hbm, out_hbm, xbuf, tbuf, cnt, srv, wb, rb,
               sp_cnt, sp_sr, sp_max, sp_tot, sp_loss):
    cid = lax.axis_index("c")
    sid = lax.axis_index("s")
    sample = cid * 8 + sid // 2
    h = sid % 2
    base = h * HALF
    iov = lax.broadcasted_iota(jnp.int32, (L,), 0)

    m = jnp.full((L,), -3.0e38, jnp.float32)
    for i in range(NCH):
        off = base + i * CH
        pltpu.sync_copy(x_hbm.at[sample, pl.ds(off, CH)], xbuf)
        pltpu.sync_copy(t_hbm.at[sample, pl.ds(off, CH)], tbuf)

        def vmax_body(j, mm):
            xx = xbuf[pl.ds(j * L, L)]
            tf = tbuf[pl.ds(j * L, L)].astype(jnp.float32)
            e = 1.0 - xx * (2.0 * tf - 1.0)
            return jnp.maximum(mm, e)

        m = lax.fori_loop(0, CH // L, vmax_body, m)

    wb[...] = m
    pltpu.sync_copy(wb, sp_max.at[pl.ds(sid * L, L)])
    plsc.subcore_barrier()
    pltpu.sync_copy(sp_max.at[pl.ds((sid ^ 1) * L, L)], rb)
    emax = jnp.max(jnp.maximum(m, rb[...]))

    zi = jnp.zeros((L,), jnp.int32)
    zf = jnp.zeros((L,), jnp.float32)

    def zc_body(j, c):
        cnt[pl.ds(j * L, L)] = zi
        return c

    lax.fori_loop(0, (2 * NB) // L, zc_body, 0)

    def zs_body(j, c):
        srv[pl.ds(j * L, L)] = zf
        return c

    lax.fori_loop(0, NB // L, zs_body, 0)

    emaxv = jnp.zeros((L,), jnp.float32) + emax
    scale = NB / jnp.maximum(emaxv, 1e-30)
    onei = jnp.ones((L,), jnp.int32)
    for i in range(NCH):
        off = base + i * CH
        pltpu.sync_copy(x_hbm.at[sample, pl.ds(off, CH)], xbuf)
        pltpu.sync_copy(t_hbm.at[sample, pl.ds(off, CH)], tbuf)

        def scat_body(j, c):
            xx = xbuf[pl.ds(j * L, L)]
            ti = tbuf[pl.ds(j * L, L)]
            tf = ti.astype(jnp.float32)
            e = 1.0 - xx * (2.0 * tf - 1.0)
            relu = jnp.maximum(e, 0.0)
            bf = jnp.clip((emax - e) * scale, 0.0, float(NB - 1))
            b = bf.astype(jnp.int32)
            plsc.addupdate_scatter(cnt, [b + (1 - ti) * NB], onei)
            plsc.addupdate_scatter(srv, [b], relu)
            return c

        lax.fori_loop(0, CH // L, scat_body, 0)

    pltpu.sync_copy(cnt, sp_cnt.at[pl.ds(sid * 2 * NB, 2 * NB)])
    pltpu.sync_copy(srv, sp_sr.at[pl.ds(sid * NB, NB)])
    plsc.subcore_barrier()

    s0 = h * HB
    pbase_c = (sid ^ 1) * 2 * NB
    pbase_s = (sid ^ 1) * NB

    pltpu.sync_copy(sp_cnt.at[pl.ds(pbase_c + s0, HB)], tbuf)

    def addcp_body(j, acc):
        o = s0 + j * L
        v = cnt[pl.ds(o, L)] + tbuf[pl.ds(j * L, L)]
        cnt[pl.ds(o, L)] = v
        return acc + v

    accp = lax.fori_loop(0, HB // L, addcp_body, zi)
    tcp = jnp.sum(accp).astype(jnp.float32)

    pltpu.sync_copy(sp_cnt.at[pl.ds(pbase_c + NB + s0, HB)], tbuf)

    def addcn_body(j, acc):
        o = NB + s0 + j * L
        v = cnt[pl.ds(o, L)] + tbuf[pl.ds(j * L, L)]
        cnt[pl.ds(o, L)] = v
        return acc + v

    accn = lax.fori_loop(0, HB // L, addcn_body, zi)
    tcn = jnp.sum(accn).astype(jnp.float32)

    pltpu.sync_copy(sp_sr.at[pl.ds(pbase_s + s0, HB)], xbuf)

    def addsr_body(j, c):
        o = s0 + j * L
        srv[pl.ds(o, L)] = srv[pl.ds(o, L)] + xbuf[pl.ds(j * L, L)]
        return c

    lax.fori_loop(0, HB // L, addsr_body, 0)

    wb[...] = jnp.where(iov == 0, tcp, jnp.where(iov == 1, tcn, 0.0))
    pltpu.sync_copy(wb, sp_tot.at[pl.ds(sid * L, L)])
    plsc.subcore_barrier()
    pltpu.sync_copy(sp_tot.at[pl.ds((sid ^ 1) * L, L)], rb)
    pv = rb[...]
    tcp_p = jnp.sum(jnp.where(iov == 0, pv, 0.0))
    tcn_p = jnp.sum(jnp.where(iov == 1, pv, 0.0))
    g = tcp + tcp_p
    hf = h.astype(jnp.float32)
    base_a = hf * tcp_p
    base_b = hf * tcn_p

    def scan_body(j, carry):
        sa, sb, acc = carry
        o = s0 + j * L
        cpf = cnt[pl.ds(o, L)].astype(jnp.float32)
        cnf = cnt[pl.ds(NB + o, L)].astype(jnp.float32)
        sr16 = srv[pl.ds(o, L)]
        icp = plsc.cumsum(cpf)
        icn = plsc.cumsum(cnf)
        ea = sa + icp - cpf
        eb = sb + icn - cnf
        pe = ea + cpf
        ne = eb + cnf
        d1 = g + eb
        d2 = g + ne
        j1 = jnp.where(d1 > 0, 1.0 - (g - ea) / jnp.maximum(d1, 1.0), 0.0)
        j2 = jnp.where(d2 > 0, 1.0 - (g - pe) / jnp.maximum(d2, 1.0), 0.0)
        cc = cpf + cnf
        contrib = jnp.where(cc > 0, sr16 * (j2 - j1) / jnp.maximum(cc, 1.0), 0.0)
        return sa + jnp.sum(cpf), sb + jnp.sum(cnf), acc + contrib

    _, _, acc = lax.fori_loop(0, HB // L, scan_body, (base_a, base_b, zf))

    wb[...] = acc
    pltpu.sync_copy(wb, sp_loss.at[pl.ds(sid * L, L)])
    plsc.subcore_barrier()
    pltpu.sync_copy(sp_loss.at[pl.ds((sid ^ 1) * L, L)], rb)
    loss = jnp.sum(acc + rb[...])

    @pl.when(h == 0)
    def _():
        wb[...] = jnp.where(iov == 0, loss, 0.0)
        pltpu.sync_copy(wb, out_hbm.at[sample])


def kernel(inputs, targets):
    x2 = inputs.reshape(S, P)
    t2 = targets.astype(jnp.int32).reshape(S, P)
    out = _lovasz_sc(x2, t2)
    return jnp.mean(out[:, 0])


# speedup vs baseline: 14.7540x; 1.1855x over previous
import functools

import jax
import jax.numpy as jnp
from jax import lax
from jax.experimental import pallas as pl
from jax.experimental.pallas import tpu as pltpu
from jax.experimental.pallas import tpu_sc as plsc

S = 16
P = 512 * 512
HALF = P // 2
NB = 16384
HB = NB // 2
CH = 8192
NCH = HALF // CH
L = 16
U = 8

_mesh = plsc.VectorSubcoreMesh(core_axis_name="c", subcore_axis_name="s")


@functools.partial(
    pl.kernel,
    out_type=jax.ShapeDtypeStruct((S, L), jnp.float32),
    mesh=_mesh,
    compiler_params=pltpu.CompilerParams(needs_layout_passes=False),
    scratch_types=[
        pltpu.VMEM((CH,), jnp.float32),
        pltpu.VMEM((CH,), jnp.int32),
        pltpu.VMEM((2 * NB,), jnp.int32),
        pltpu.VMEM((NB,), jnp.float32),
        pltpu.VMEM((L,), jnp.float32),
        pltpu.VMEM((L,), jnp.float32),
        pltpu.VMEM_SHARED((16 * 2 * NB,), jnp.int32),
        pltpu.VMEM_SHARED((16 * NB,), jnp.float32),
        pltpu.VMEM_SHARED((16 * L,), jnp.float32),
        pltpu.VMEM_SHARED((16 * L,), jnp.float32),
        pltpu.VMEM_SHARED((16 * L,), jnp.float32),
    ],
)
def _lovasz_sc(x_hbm, t_hbm, out_hbm, xbuf, tbuf, cnt, srv, wb, rb,
               sp_cnt, sp_sr, sp_max, sp_tot, sp_loss):
    cid = lax.axis_index("c")
    sid = lax.axis_index("s")
    sample = cid * 8 + sid // 2
    h = sid % 2
    base = h * HALF
    iov = lax.broadcasted_iota(jnp.int32, (L,), 0)
    zi = jnp.zeros((L,), jnp.int32)
    zf = jnp.zeros((L,), jnp.float32)

    def p1_chunk(i, m):
        pltpu.sync_copy(x_hbm.at[sample, pl.ds(base + i * CH, CH)], xbuf)

        def p1v(j, mm):
            for k in range(U):
                xx = xbuf[pl.ds((j * U + k) * L, L)]
                mm = jnp.maximum(mm, jnp.abs(xx))
            return mm

        return lax.fori_loop(0, CH // (L * U), p1v, m)

    m = lax.fori_loop(0, NCH, p1_chunk, zf)

    wb[...] = m
    pltpu.sync_copy(wb, sp_max.at[pl.ds(sid * L, L)])
    plsc.subcore_barrier()
    pltpu.sync_copy(sp_max.at[pl.ds((sid ^ 1) * L, L)], rb)
    emax = 1.0 + jnp.max(jnp.maximum(m, rb[...]))

    def zc_body(j, c):
        for k in range(2 * U):
            cnt[pl.ds((j * 2 * U + k) * L, L)] = zi
        return c

    lax.fori_loop(0, (2 * NB) // (L * 2 * U), zc_body, 0)

    def zs_body(j, c):
        for k in range(2 * U):
            srv[pl.ds((j * 2 * U + k) * L, L)] = zf
        return c

    lax.fori_loop(0, NB // (L * 2 * U), zs_body, 0)

    emaxv = zf + emax
    scale = NB / jnp.maximum(emaxv, 1e-30)
    onei = jnp.ones((L,), jnp.int32)
    nbv = jnp.full((L,), NB, jnp.int32)

    def p2_chunk(i, c):
        off = base + i * CH
        pltpu.sync_copy(x_hbm.at[sample, pl.ds(off, CH)], xbuf)
        pltpu.sync_copy(t_hbm.at[sample, pl.ds(off, CH)], tbuf)

        def p2v(j, cc):
            for k in range(U):
                o = (j * U + k) * L
                xx = xbuf[pl.ds(o, L)]
                ti = tbuf[pl.ds(o, L)]
                tf = ti.astype(jnp.float32)
                e = 1.0 - xx * (2.0 * tf - 1.0)
                relu = jnp.maximum(e, 0.0)
                bf = jnp.clip((emaxv - e) * scale, 0.0, float(NB - 1))
                b = bf.astype(jnp.int32)
                plsc.addupdate_scatter(cnt, [b + (nbv - ti * NB)], onei)
                plsc.addupdate_scatter(srv, [b], relu)
            return cc

        return lax.fori_loop(0, CH // (L * U), p2v, c)

    lax.fori_loop(0, NCH, p2_chunk, 0)

    pltpu.sync_copy(cnt, sp_cnt.at[pl.ds(sid * 2 * NB, 2 * NB)])
    pltpu.sync_copy(srv, sp_sr.at[pl.ds(sid * NB, NB)])
    plsc.subcore_barrier()

    s0 = h * HB
    pbase_c = (sid ^ 1) * 2 * NB
    pbase_s = (sid ^ 1) * NB

    pltpu.sync_copy(sp_cnt.at[pl.ds(pbase_c + s0, HB)], tbuf)

    def addcp_body(j, acc):
        for k in range(U):
            o = (j * U + k) * L
            v = cnt[pl.ds(s0 + o, L)] + tbuf[pl.ds(o, L)]
            cnt[pl.ds(s0 + o, L)] = v
            acc = acc + v
        return acc

    accp = lax.fori_loop(0, HB // (L * U), addcp_body, zi)
    tcp = jnp.sum(accp).astype(jnp.float32)

    pltpu.sync_copy(sp_cnt.at[pl.ds(pbase_c + NB + s0, HB)], tbuf)

    def addcn_body(j, acc):
        for k in range(U):
            o = (j * U + k) * L
            v = cnt[pl.ds(NB + s0 + o, L)] + tbuf[pl.ds(o, L)]
            cnt[pl.ds(NB + s0 + o, L)] = v
            acc = acc + v
        return acc

    accn = lax.fori_loop(0, HB // (L * U), addcn_body, zi)
    tcn = jnp.sum(accn).astype(jnp.float32)

    pltpu.sync_copy(sp_sr.at[pl.ds(pbase_s + s0, HB)], xbuf)

    def addsr_body(j, c):
        for k in range(U):
            o = (j * U + k) * L
            srv[pl.ds(s0 + o, L)] = srv[pl.ds(s0 + o, L)] + xbuf[pl.ds(o, L)]
        return c

    lax.fori_loop(0, HB // (L * U), addsr_body, 0)

    wb[...] = jnp.where(iov == 0, tcp, jnp.where(iov == 1, tcn, 0.0))
    pltpu.sync_copy(wb, sp_tot.at[pl.ds(sid * L, L)])
    plsc.subcore_barrier()
    pltpu.sync_copy(sp_tot.at[pl.ds((sid ^ 1) * L, L)], rb)
    pv = rb[...]
    tcp_p = jnp.sum(jnp.where(iov == 0, pv, 0.0))
    tcn_p = jnp.sum(jnp.where(iov == 1, pv, 0.0))
    g = tcp + tcp_p
    hf = h.astype(jnp.float32)
    base_a = zf + hf * tcp_p
    base_b = zf + hf * tcn_p

    def scan_body(j, carry):
        sav, sbv, acc = carry
        o = s0 + j * L
        cpf = cnt[pl.ds(o, L)].astype(jnp.float32)
        cnf = cnt[pl.ds(NB + o, L)].astype(jnp.float32)
        sr16 = srv[pl.ds(o, L)]
        icp = plsc.cumsum(cpf)
        icn = plsc.cumsum(cnf)
        ea = sav + icp - cpf
        eb = sbv + icn - cnf
        pe = ea + cpf
        ne = eb + cnf
        d1 = g + eb
        d2 = g + ne
        j1 = jnp.where(d1 > 0, 1.0 - (g - ea) / jnp.maximum(d1, 1.0), 0.0)
        j2 = jnp.where(d2 > 0, 1.0 - (g - pe) / jnp.maximum(d2, 1.0), 0.0)
        cc = cpf + cnf
        contrib = jnp.where(cc > 0, sr16 * (j2 - j1) / jnp.maximum(cc, 1.0), 0.0)
        sav = sav + jnp.sum(cpf)
        sbv = sbv + jnp.sum(cnf)
        return sav, sbv, acc + contrib

    _, _, acc = lax.fori_loop(0, HB // L, scan_body, (base_a, base_b, zf))

    wb[...] = acc
    pltpu.sync_copy(wb, sp_loss.at[pl.ds(sid * L, L)])
    plsc.subcore_barrier()
    pltpu.sync_copy(sp_loss.at[pl.ds((sid ^ 1) * L, L)], rb)
    loss = jnp.sum(acc + rb[...])

    @pl.when(h == 0)
    def _():
        wb[...] = jnp.where(iov == 0, loss, 0.0)
        pltpu.sync_copy(wb, out_hbm.at[sample])


def kernel(inputs, targets):
    x2 = inputs.reshape(S, P)
    t2 = targets.astype(jnp.int32).reshape(S, P)
    out = _lovasz_sc(x2, t2)
    return jnp.mean(out[:, 0])


# speedup vs baseline: 16.5101x; 1.1190x over previous
import functools

import jax
import jax.numpy as jnp
from jax import lax
from jax.experimental import pallas as pl
from jax.experimental.pallas import tpu as pltpu
from jax.experimental.pallas import tpu_sc as plsc

S = 16
P = 512 * 512
HALF = P // 2
NB = 16384
HB = NB // 2
CH = 8192
NCH = HALF // CH
L = 16
U = 8

_mesh = plsc.VectorSubcoreMesh(core_axis_name="c", subcore_axis_name="s")


@functools.partial(
    pl.kernel,
    out_type=jax.ShapeDtypeStruct((S * L,), jnp.float32),
    mesh=_mesh,
    compiler_params=pltpu.CompilerParams(needs_layout_passes=False),
    scratch_types=[
        pltpu.VMEM((CH,), jnp.float32),
        pltpu.VMEM((CH,), jnp.float32),
        pltpu.VMEM((CH,), jnp.int32),
        pltpu.VMEM((CH,), jnp.int32),
        pltpu.VMEM((2 * NB,), jnp.int32),
        pltpu.VMEM((NB,), jnp.float32),
        pltpu.VMEM((L,), jnp.float32),
        pltpu.VMEM((L,), jnp.float32),
        pltpu.VMEM_SHARED((16 * 2 * HB,), jnp.int32),
        pltpu.VMEM_SHARED((16 * HB,), jnp.float32),
        pltpu.VMEM_SHARED((16 * L,), jnp.float32),
        pltpu.VMEM_SHARED((16 * L,), jnp.float32),
        pltpu.VMEM_SHARED((16 * L,), jnp.float32),
        pltpu.SemaphoreType.DMA,
        pltpu.SemaphoreType.DMA,
        pltpu.SemaphoreType.DMA,
        pltpu.SemaphoreType.DMA,
    ],
)
def _lovasz_sc(x_hbm, t_hbm, out_hbm, xb0, xb1, tb0, tb1, cnt, srv, wb, rb,
               sp_cnt, sp_sr, sp_max, sp_tot, sp_loss,
               smx0, smx1, smt0, smt1):
    cid = lax.axis_index("c")
    sid = lax.axis_index("s")
    sample = cid * 8 + sid // 2
    h = sid % 2
    base = sample * P + h * HALF
    iov = lax.broadcasted_iota(jnp.int32, (L,), 0)
    zi = jnp.zeros((L,), jnp.int32)
    zf = jnp.zeros((L,), jnp.float32)
    xb = [xb0, xb1]
    tb = [tb0, tb1]
    smx = [smx0, smx1]
    smt = [smt0, smt1]

    def xsrc(i):
        return x_hbm.at[pl.ds(base + i * CH, CH)]

    def tsrc(i):
        return t_hbm.at[pl.ds(base + i * CH, CH)]

    pltpu.async_copy(xsrc(0), xb0, smx0)
    m = zf
    for i in range(NCH):
        p = i % 2
        if i + 1 < NCH:
            pltpu.async_copy(xsrc(i + 1), xb[1 - p], smx[1 - p])
        pltpu.make_async_copy(xsrc(i), xb[p], smx[p]).wait()
        buf = xb[p]

        def p1v(j, mm, buf=buf):
            for k in range(U):
                xx = buf[pl.ds((j * U + k) * L, L)]
                mm = jnp.maximum(mm, jnp.abs(xx))
            return mm

        m = lax.fori_loop(0, CH // (L * U), p1v, m)

    pltpu.async_copy(xsrc(0), xb0, smx0)
    pltpu.async_copy(tsrc(0), tb0, smt0)

    wb[...] = m
    pltpu.sync_copy(wb, sp_max.at[pl.ds(sid * L, L)])
    plsc.subcore_barrier()
    pltpu.sync_copy(sp_max.at[pl.ds((sid ^ 1) * L, L)], rb)
    emax = 1.0 + jnp.max(jnp.maximum(m, rb[...]))

    def zc_body(j, c):
        for k in range(2 * U):
            cnt[pl.ds((j * 2 * U + k) * L, L)] = zi
        return c

    lax.fori_loop(0, (2 * NB) // (L * 2 * U), zc_body, 0)

    def zs_body(j, c):
        for k in range(2 * U):
            srv[pl.ds((j * 2 * U + k) * L, L)] = zf
        return c

    lax.fori_loop(0, NB // (L * 2 * U), zs_body, 0)

    emaxv = zf + emax
    scale = NB / jnp.maximum(emaxv, 1e-30)
    onei = jnp.ones((L,), jnp.int32)
    nbv = jnp.full((L,), NB, jnp.int32)

    for i in range(NCH):
        p = i % 2
        if i + 1 < NCH:
            pltpu.async_copy(xsrc(i + 1), xb[1 - p], smx[1 - p])
            pltpu.async_copy(tsrc(i + 1), tb[1 - p], smt[1 - p])
        pltpu.make_async_copy(xsrc(i), xb[p], smx[p]).wait()
        pltpu.make_async_copy(tsrc(i), tb[p], smt[p]).wait()
        bx, bt = xb[p], tb[p]

        def p2v(j, cc, bx=bx, bt=bt):
            for k in range(U):
                o = (j * U + k) * L
                xx = bx[pl.ds(o, L)]
                ti = bt[pl.ds(o, L)]
                tf = ti.astype(jnp.float32)
                e = 1.0 - xx * (2.0 * tf - 1.0)
                relu = jnp.maximum(e, 0.0)
                bf = jnp.clip((emaxv - e) * scale, 0.0, float(NB - 1))
                b = bf.astype(jnp.int32)
                plsc.addupdate_scatter(cnt, [b + (nbv - ti * NB)], onei)
                plsc.addupdate_scatter(srv, [b], relu)
            return cc

        lax.fori_loop(0, CH // (L * U), p2v, 0)

    s0 = h * HB
    ps0 = HB - s0
    pltpu.sync_copy(cnt.at[pl.ds(ps0, HB)],
                    sp_cnt.at[pl.ds(sid * 2 * HB, HB)])
    pltpu.sync_copy(cnt.at[pl.ds(NB + ps0, HB)],
                    sp_cnt.at[pl.ds(sid * 2 * HB + HB, HB)])
    pltpu.sync_copy(srv.at[pl.ds(ps0, HB)],
                    sp_sr.at[pl.ds(sid * HB, HB)])
    plsc.subcore_barrier()

    pbase_c = (sid ^ 1) * 2 * HB
    pbase_s = (sid ^ 1) * HB

    pltpu.async_copy(sp_cnt.at[pl.ds(pbase_c, HB)], tb0.at[pl.ds(0, HB)], smt0)
    pltpu.async_copy(sp_cnt.at[pl.ds(pbase_c + HB, HB)], tb1.at[pl.ds(0, HB)], smt1)
    pltpu.async_copy(sp_sr.at[pl.ds(pbase_s, HB)], xb0.at[pl.ds(0, HB)], smx0)

    pltpu.make_async_copy(sp_cnt.at[pl.ds(pbase_c, HB)], tb0.at[pl.ds(0, HB)], smt0).wait()

    def addcp_body(j, acc):
        for k in range(U):
            o = (j * U + k) * L
            v = cnt[pl.ds(s0 + o, L)] + tb0[pl.ds(o, L)]
            cnt[pl.ds(s0 + o, L)] = v
            acc = acc + v
        return acc

    accp = lax.fori_loop(0, HB // (L * U), addcp_body, zi)
    tcp = jnp.sum(accp).astype(jnp.float32)

    pltpu.make_async_copy(sp_cnt.at[pl.ds(pbase_c + HB, HB)], tb1.at[pl.ds(0, HB)], smt1).wait()

    def addcn_body(j, acc):
        for k in range(U):
            o = (j * U + k) * L
            v = cnt[pl.ds(NB + s0 + o, L)] + tb1[pl.ds(o, L)]
            cnt[pl.ds(NB + s0 + o, L)] = v
            acc = acc + v
        return acc

    accn = lax.fori_loop(0, HB // (L * U), addcn_body, zi)
    tcn = jnp.sum(accn).astype(jnp.float32)

    pltpu.make_async_copy(sp_sr.at[pl.ds(pbase_s, HB)], xb0.at[pl.ds(0, HB)], smx0).wait()

    def addsr_body(j, c):
        for k in range(U):
            o = (j * U + k) * L
            srv[pl.ds(s0 + o, L)] = srv[pl.ds(s0 + o, L)] + xb0[pl.ds(o, L)]
        return c

    lax.fori_loop(0, HB // (L * U), addsr_body, 0)

    wb[...] = jnp.where(iov == 0, tcp, jnp.where(iov == 1, tcn, 0.0))
    pltpu.sync_copy(wb, sp_tot.at[pl.ds(sid * L, L)])
    plsc.subcore_barrier()
    pltpu.sync_copy(sp_tot.at[pl.ds((sid ^ 1) * L, L)], rb)
    pv = rb[...]
    tcp_p = jnp.sum(jnp.where(iov == 0, pv, 0.0))
    tcn_p = jnp.sum(jnp.where(iov == 1, pv, 0.0))
    g = tcp + tcp_p
    hf = h.astype(jnp.float32)
    base_a = zf + hf * tcp_p
    base_b = zf + hf * tcn_p

    def scan_body(j, carry):
        sav, sbv, acc = carry
        o = s0 + j * L
        cpf = cnt[pl.ds(o, L)].astype(jnp.float32)
        cnf = cnt[pl.ds(NB + o, L)].astype(jnp.float32)
        sr16 = srv[pl.ds(o, L)]
        icp = plsc.cumsum(cpf)
        icn = plsc.cumsum(cnf)
        ea = sav + icp - cpf
        eb = sbv + icn - cnf
        pe = ea + cpf
        ne = eb + cnf
        d1 = g + eb
        d2 = g + ne
        j1 = jnp.where(d1 > 0, 1.0 - (g - ea) / jnp.maximum(d1, 1.0), 0.0)
        j2 = jnp.where(d2 > 0, 1.0 - (g - pe) / jnp.maximum(d2, 1.0), 0.0)
        cc = cpf + cnf
        contrib = jnp.where(cc > 0, sr16 * (j2 - j1) / jnp.maximum(cc, 1.0), 0.0)
        sav = sav + jnp.sum(cpf)
        sbv = sbv + jnp.sum(cnf)
        return sav, sbv, acc + contrib

    _, _, acc = lax.fori_loop(0, HB // L, scan_body, (base_a, base_b, zf))

    wb[...] = acc
    pltpu.sync_copy(wb, sp_loss.at[pl.ds(sid * L, L)])
    plsc.subcore_barrier()
    pltpu.sync_copy(sp_loss.at[pl.ds((sid ^ 1) * L, L)], rb)
    loss = jnp.sum(acc + rb[...])

    @pl.when(h == 0)
    def _():
        wb[...] = jnp.where(iov == 0, loss, 0.0)
        pltpu.sync_copy(wb, out_hbm.at[pl.ds(sample * L, L)])


def kernel(inputs, targets):
    x1 = inputs.reshape(S * P)
    t1 = targets.astype(jnp.int32).reshape(S * P)
    out = _lovasz_sc(x1, t1)
    return jnp.mean(out.reshape(S, L)[:, 0])


# speedup vs baseline: 18.1209x; 1.0976x over previous
import functools

import jax
import jax.numpy as jnp
from jax import lax
from jax.experimental import pallas as pl
from jax.experimental.pallas import tpu as pltpu
from jax.experimental.pallas import tpu_sc as plsc

S = 16
P = 512 * 512
HALF = P // 2
NB = 16384
HB = NB // 2
CH = 8192
NCH = HALF // CH
L = 16
U = 16

_mesh = plsc.VectorSubcoreMesh(core_axis_name="c", subcore_axis_name="s")


@functools.partial(
    pl.kernel,
    out_type=jax.ShapeDtypeStruct((S * L,), jnp.float32),
    mesh=_mesh,
    compiler_params=pltpu.CompilerParams(needs_layout_passes=False),
    scratch_types=[
        pltpu.VMEM((CH,), jnp.float32),
        pltpu.VMEM((CH,), jnp.float32),
        pltpu.VMEM((CH,), jnp.int32),
        pltpu.VMEM((CH,), jnp.int32),
        pltpu.VMEM((2 * NB,), jnp.int32),
        pltpu.VMEM((NB,), jnp.float32),
        pltpu.VMEM((L,), jnp.float32),
        pltpu.VMEM((L,), jnp.float32),
        pltpu.VMEM_SHARED((16 * 2 * HB,), jnp.int32),
        pltpu.VMEM_SHARED((16 * HB,), jnp.float32),
        pltpu.VMEM_SHARED((16 * L,), jnp.float32),
        pltpu.VMEM_SHARED((16 * L,), jnp.float32),
        pltpu.VMEM_SHARED((16 * L,), jnp.float32),
        pltpu.SemaphoreType.DMA,
        pltpu.SemaphoreType.DMA,
        pltpu.SemaphoreType.DMA,
        pltpu.SemaphoreType.DMA,
    ],
)
def _lovasz_sc(x_hbm, t_hbm, out_hbm, xb0, xb1, tb0, tb1, cnt, srv, wb, rb,
               sp_cnt, sp_sr, sp_max, sp_tot, sp_loss,
               smx0, smx1, smt0, smt1):
    cid = lax.axis_index("c")
    sid = lax.axis_index("s")
    sample = cid * 8 + sid // 2
    h = sid % 2
    base = sample * P + h * HALF
    iov = lax.broadcasted_iota(jnp.int32, (L,), 0)
    zi = jnp.zeros((L,), jnp.int32)
    zf = jnp.zeros((L,), jnp.float32)
    xb = [xb0, xb1]
    tb = [tb0, tb1]
    smx = [smx0, smx1]
    smt = [smt0, smt1]

    def xsrc(i):
        return x_hbm.at[pl.ds(base + i * CH, CH)]

    def tsrc(i):
        return t_hbm.at[pl.ds(base + i * CH, CH)]

    pltpu.async_copy(xsrc(0), xb0, smx0)
    m = zf
    for i in range(NCH):
        p = i % 2
        if i + 1 < NCH:
            pltpu.async_copy(xsrc(i + 1), xb[1 - p], smx[1 - p])
        pltpu.make_async_copy(xsrc(i), xb[p], smx[p]).wait()
        buf = xb[p]

        def p1v(j, mm, buf=buf):
            for k in range(U):
                xx = buf[pl.ds((j * U + k) * L, L)]
                mm = jnp.maximum(mm, jnp.abs(xx))
            return mm

        m = lax.fori_loop(0, CH // (L * U), p1v, m)

    pltpu.async_copy(xsrc(0), xb0, smx0)
    pltpu.async_copy(tsrc(0), tb0, smt0)

    wb[...] = m
    pltpu.sync_copy(wb, sp_max.at[pl.ds(sid * L, L)])
    plsc.subcore_barrier()
    pltpu.sync_copy(sp_max.at[pl.ds((sid ^ 1) * L, L)], rb)
    emax = 1.0 + jnp.max(jnp.maximum(m, rb[...]))

    def zc_body(j, c):
        for k in range(2 * U):
            cnt[pl.ds((j * 2 * U + k) * L, L)] = zi
        return c

    lax.fori_loop(0, (2 * NB) // (L * 2 * U), zc_body, 0)

    def zs_body(j, c):
        for k in range(2 * U):
            srv[pl.ds((j * 2 * U + k) * L, L)] = zf
        return c

    lax.fori_loop(0, NB // (L * 2 * U), zs_body, 0)

    emaxv = zf + emax
    scale = NB / jnp.maximum(emaxv, 1e-30)
    c0s = (emaxv - 1.0) * scale
    bmax = jnp.full((L,), float(NB - 1), jnp.float32)
    onei = jnp.ones((L,), jnp.int32)

    for i in range(NCH):
        p = i % 2
        if i + 1 < NCH:
            pltpu.async_copy(xsrc(i + 1), xb[1 - p], smx[1 - p])
            pltpu.async_copy(tsrc(i + 1), tb[1 - p], smt[1 - p])
        pltpu.make_async_copy(xsrc(i), xb[p], smx[p]).wait()
        pltpu.make_async_copy(tsrc(i), tb[p], smt[p]).wait()
        bx, bt = xb[p], tb[p]

        def p2v(j, cc, bx=bx, bt=bt):
            for k in range(U):
                o = (j * U + k) * L
                xx = bx[pl.ds(o, L)]
                ti = bt[pl.ds(o, L)]
                tf = ti.astype(jnp.float32)
                xs = xx * (2.0 * tf - 1.0)
                relu = jnp.maximum(1.0 - xs, 0.0)
                bf = jnp.minimum(c0s + xs * scale, bmax)
                b = bf.astype(jnp.int32)
                plsc.addupdate_scatter(cnt, [b + ti * NB], onei)
                plsc.addupdate_scatter(srv, [b], relu)
            return cc

        lax.fori_loop(0, CH // (L * U), p2v, 0)

    s0 = h * HB
    ps0 = HB - s0
    pltpu.sync_copy(cnt.at[pl.ds(ps0, HB)],
                    sp_cnt.at[pl.ds(sid * 2 * HB, HB)])
    pltpu.sync_copy(cnt.at[pl.ds(NB + ps0, HB)],
                    sp_cnt.at[pl.ds(sid * 2 * HB + HB, HB)])
    pltpu.sync_copy(srv.at[pl.ds(ps0, HB)],
                    sp_sr.at[pl.ds(sid * HB, HB)])
    plsc.subcore_barrier()

    pbase_c = (sid ^ 1) * 2 * HB
    pbase_s = (sid ^ 1) * HB

    pltpu.async_copy(sp_cnt.at[pl.ds(pbase_c, HB)], tb0.at[pl.ds(0, HB)], smt0)
    pltpu.async_copy(sp_cnt.at[pl.ds(pbase_c + HB, HB)], tb1.at[pl.ds(0, HB)], smt1)
    pltpu.async_copy(sp_sr.at[pl.ds(pbase_s, HB)], xb0.at[pl.ds(0, HB)], smx0)

    pltpu.make_async_copy(sp_cnt.at[pl.ds(pbase_c, HB)], tb0.at[pl.ds(0, HB)], smt0).wait()

    def addcn_body(j, acc):
        for k in range(U):
            o = (j * U + k) * L
            v = cnt[pl.ds(s0 + o, L)] + tb0[pl.ds(o, L)]
            cnt[pl.ds(s0 + o, L)] = v
            acc = acc + v
        return acc

    accn = lax.fori_loop(0, HB // (L * U), addcn_body, zi)
    tcn = jnp.sum(accn).astype(jnp.float32)

    pltpu.make_async_copy(sp_cnt.at[pl.ds(pbase_c + HB, HB)], tb1.at[pl.ds(0, HB)], smt1).wait()

    def addcp_body(j, acc):
        for k in range(U):
            o = (j * U + k) * L
            v = cnt[pl.ds(NB + s0 + o, L)] + tb1[pl.ds(o, L)]
            cnt[pl.ds(NB + s0 + o, L)] = v
            acc = acc + v
        return acc

    accp = lax.fori_loop(0, HB // (L * U), addcp_body, zi)
    tcp = jnp.sum(accp).astype(jnp.float32)

    pltpu.make_async_copy(sp_sr.at[pl.ds(pbase_s, HB)], xb0.at[pl.ds(0, HB)], smx0).wait()

    def addsr_body(j, c):
        for k in range(U):
            o = (j * U + k) * L
            srv[pl.ds(s0 + o, L)] = srv[pl.ds(s0 + o, L)] + xb0[pl.ds(o, L)]
        return c

    lax.fori_loop(0, HB // (L * U), addsr_body, 0)

    wb[...] = jnp.where(iov == 0, tcp, jnp.where(iov == 1, tcn, 0.0))
    pltpu.sync_copy(wb, sp_tot.at[pl.ds(sid * L, L)])
    plsc.subcore_barrier()
    pltpu.sync_copy(sp_tot.at[pl.ds((sid ^ 1) * L, L)], rb)
    pv = rb[...]
    tcp_p = jnp.sum(jnp.where(iov == 0, pv, 0.0))
    tcn_p = jnp.sum(jnp.where(iov == 1, pv, 0.0))
    g = tcp + tcp_p
    hf = h.astype(jnp.float32)
    base_a = zf + hf * tcp_p
    base_b = zf + hf * tcn_p

    def scan_body(j, carry):
        sav, sbv, acc = carry
        for k in range(2):
            o = s0 + (j * 2 + k) * L
            cpf = cnt[pl.ds(NB + o, L)].astype(jnp.float32)
            cnf = cnt[pl.ds(o, L)].astype(jnp.float32)
            sr16 = srv[pl.ds(o, L)]
            icp = plsc.cumsum(cpf)
            icn = plsc.cumsum(cnf)
            ea = sav + icp - cpf
            eb = sbv + icn - cnf
            pe = ea + cpf
            ne = eb + cnf
            d1 = g + eb
            d2 = g + ne
            j1 = jnp.where(d1 > 0, 1.0 - (g - ea) / jnp.maximum(d1, 1.0), 0.0)
            j2 = jnp.where(d2 > 0, 1.0 - (g - pe) / jnp.maximum(d2, 1.0), 0.0)
            cc = cpf + cnf
            contrib = jnp.where(cc > 0, sr16 * (j2 - j1) / jnp.maximum(cc, 1.0), 0.0)
            sav = sav + jnp.sum(cpf)
            sbv = sbv + jnp.sum(cnf)
            acc = acc + contrib
        return sav, sbv, acc

    _, _, acc = lax.fori_loop(0, HB // (L * 2), scan_body, (base_a, base_b, zf))

    wb[...] = acc
    pltpu.sync_copy(wb, sp_loss.at[pl.ds(sid * L, L)])
    plsc.subcore_barrier()
    pltpu.sync_copy(sp_loss.at[pl.ds((sid ^ 1) * L, L)], rb)
    loss = jnp.sum(acc + rb[...])

    @pl.when(h == 0)
    def _():
        wb[...] = jnp.where(iov == 0, loss, 0.0)
        pltpu.sync_copy(wb, out_hbm.at[pl.ds(sample * L, L)])


def kernel(inputs, targets):
    x1 = inputs.reshape(S * P)
    t1 = targets.astype(jnp.int32).reshape(S * P)
    out = _lovasz_sc(x1, t1)
    return jnp.mean(out.reshape(S, L)[:, 0])


# speedup vs baseline: 32.1327x; 1.7732x over previous
import functools

import jax
import jax.numpy as jnp
from jax import lax
from jax.experimental import pallas as pl
from jax.experimental.pallas import tpu as pltpu
from jax.experimental.pallas import tpu_sc as plsc

S = 16
P = 512 * 512
HALF = P // 2
NB = 16384
HB = NB // 2
CH = 8192
NCH = HALF // CH
L = 16
U = 16

_mesh = plsc.VectorSubcoreMesh(core_axis_name="c", subcore_axis_name="s")


@functools.partial(
    pl.kernel,
    out_type=jax.ShapeDtypeStruct((S * L,), jnp.float32),
    mesh=_mesh,
    compiler_params=pltpu.CompilerParams(needs_layout_passes=False),
    scratch_types=[
        pltpu.VMEM((CH,), jnp.float32),
        pltpu.VMEM((CH,), jnp.float32),
        pltpu.VMEM((CH,), jnp.int32),
        pltpu.VMEM((CH,), jnp.int32),
        pltpu.VMEM((2 * NB,), jnp.int32),
        pltpu.VMEM((NB,), jnp.float32),
        pltpu.VMEM((L,), jnp.float32),
        pltpu.VMEM((L,), jnp.float32),
        pltpu.VMEM_SHARED((16 * 2 * HB,), jnp.int32),
        pltpu.VMEM_SHARED((16 * HB,), jnp.float32),
        pltpu.VMEM_SHARED((16 * L,), jnp.float32),
        pltpu.VMEM_SHARED((16 * L,), jnp.float32),
        pltpu.VMEM_SHARED((16 * L,), jnp.float32),
        pltpu.SemaphoreType.DMA,
        pltpu.SemaphoreType.DMA,
        pltpu.SemaphoreType.DMA,
        pltpu.SemaphoreType.DMA,
    ],
)
def _lovasz_sc(x_hbm, t_hbm, out_hbm, xb0, xb1, tb0, tb1, cnt, srv, wb, rb,
               sp_cnt, sp_sr, sp_max, sp_tot, sp_loss,
               smx0, smx1, smt0, smt1):
    cid = lax.axis_index("c")
    sid = lax.axis_index("s")
    sample = cid * 8 + sid // 2
    h = sid % 2
    base = sample * P + h * HALF
    iov = lax.broadcasted_iota(jnp.int32, (L,), 0)
    zi = jnp.zeros((L,), jnp.int32)
    zf = jnp.zeros((L,), jnp.float32)
    xb = [xb0, xb1]
    tb = [tb0, tb1]
    smx = [smx0, smx1]
    smt = [smt0, smt1]

    def xsrc(i):
        return x_hbm.at[pl.ds(base + i * CH, CH)]

    def tsrc(i):
        return t_hbm.at[pl.ds(base + i * CH, CH)]

    pltpu.async_copy(xsrc(0), xb0, smx0)
    m = zf
    for i in range(NCH):
        p = i % 2
        if i + 1 < NCH:
            pltpu.async_copy(xsrc(i + 1), xb[1 - p], smx[1 - p])
        pltpu.make_async_copy(xsrc(i), xb[p], smx[p]).wait()
        buf = xb[p]

        @plsc.parallel_loop(0, CH // L, unroll=U, carry=m)
        def p1v(j, mm, buf=buf):
            return jnp.maximum(mm, jnp.abs(buf[pl.ds(j * L, L)]))

        m = p1v

    pltpu.async_copy(xsrc(0), xb0, smx0)
    pltpu.async_copy(tsrc(0), tb0, smt0)

    wb[...] = m
    pltpu.sync_copy(wb, sp_max.at[pl.ds(sid * L, L)])
    plsc.subcore_barrier()
    pltpu.sync_copy(sp_max.at[pl.ds((sid ^ 1) * L, L)], rb)
    emax = 1.0 + jnp.max(jnp.maximum(m, rb[...]))

    @plsc.parallel_loop(0, (2 * NB) // L, unroll=2 * U)
    def _zc(j):
        cnt[pl.ds(j * L, L)] = zi

    @plsc.parallel_loop(0, NB // L, unroll=2 * U)
    def _zs(j):
        srv[pl.ds(j * L, L)] = zf

    emaxv = zf + emax
    scale = NB / jnp.maximum(emaxv, 1e-30)
    c0s = (emaxv - 1.0) * scale
    bmax = jnp.full((L,), float(NB - 1), jnp.float32)
    onei = jnp.ones((L,), jnp.int32)

    for i in range(NCH):
        p = i % 2
        if i + 1 < NCH:
            pltpu.async_copy(xsrc(i + 1), xb[1 - p], smx[1 - p])
            pltpu.async_copy(tsrc(i + 1), tb[1 - p], smt[1 - p])
        pltpu.make_async_copy(xsrc(i), xb[p], smx[p]).wait()
        pltpu.make_async_copy(tsrc(i), tb[p], smt[p]).wait()
        bx, bt = xb[p], tb[p]

        @plsc.parallel_loop(0, CH // L, unroll=U)
        def p2v(j, bx=bx, bt=bt):
            o = j * L
            xx = bx[pl.ds(o, L)]
            ti = bt[pl.ds(o, L)]
            tf = ti.astype(jnp.float32)
            xs = xx * (2.0 * tf - 1.0)
            relu = jnp.maximum(1.0 - xs, 0.0)
            bf = jnp.minimum(c0s + xs * scale, bmax)
            b = bf.astype(jnp.int32)
            plsc.addupdate_scatter(cnt, [b + ti * NB], onei)
            plsc.addupdate_scatter(srv, [b], relu)

    s0 = h * HB
    ps0 = HB - s0
    pltpu.sync_copy(cnt.at[pl.ds(ps0, HB)],
                    sp_cnt.at[pl.ds(sid * 2 * HB, HB)])
    pltpu.sync_copy(cnt.at[pl.ds(NB + ps0, HB)],
                    sp_cnt.at[pl.ds(sid * 2 * HB + HB, HB)])
    pltpu.sync_copy(srv.at[pl.ds(ps0, HB)],
                    sp_sr.at[pl.ds(sid * HB, HB)])
    plsc.subcore_barrier()

    pbase_c = (sid ^ 1) * 2 * HB
    pbase_s = (sid ^ 1) * HB

    pltpu.async_copy(sp_cnt.at[pl.ds(pbase_c, HB)], tb0.at[pl.ds(0, HB)], smt0)
    pltpu.async_copy(sp_cnt.at[pl.ds(pbase_c + HB, HB)], tb1.at[pl.ds(0, HB)], smt1)
    pltpu.async_copy(sp_sr.at[pl.ds(pbase_s, HB)], xb0.at[pl.ds(0, HB)], smx0)

    pltpu.make_async_copy(sp_cnt.at[pl.ds(pbase_c, HB)], tb0.at[pl.ds(0, HB)], smt0).wait()

    def addcn_body(j, acc):
        for k in range(U):
            o = (j * U + k) * L
            v = cnt[pl.ds(s0 + o, L)] + tb0[pl.ds(o, L)]
            cnt[pl.ds(s0 + o, L)] = v
            acc = acc + v
        return acc

    accn = lax.fori_loop(0, HB // (L * U), addcn_body, zi)
    tcn = jnp.sum(accn).astype(jnp.float32)

    pltpu.make_async_copy(sp_cnt.at[pl.ds(pbase_c + HB, HB)], tb1.at[pl.ds(0, HB)], smt1).wait()

    def addcp_body(j, acc):
        for k in range(U):
            o = (j * U + k) * L
            v = cnt[pl.ds(NB + s0 + o, L)] + tb1[pl.ds(o, L)]
            cnt[pl.ds(NB + s0 + o, L)] = v
            acc = acc + v
        return acc

    accp = lax.fori_loop(0, HB // (L * U), addcp_body, zi)
    tcp = jnp.sum(accp).astype(jnp.float32)

    pltpu.make_async_copy(sp_sr.at[pl.ds(pbase_s, HB)], xb0.at[pl.ds(0, HB)], smx0).wait()

    def addsr_body(j, c):
        for k in range(U):
            o = (j * U + k) * L
            srv[pl.ds(s0 + o, L)] = srv[pl.ds(s0 + o, L)] + xb0[pl.ds(o, L)]
        return c

    lax.fori_loop(0, HB // (L * U), addsr_body, 0)

    wb[...] = jnp.where(iov == 0, tcp, jnp.where(iov == 1, tcn, 0.0))
    pltpu.sync_copy(wb, sp_tot.at[pl.ds(sid * L, L)])
    plsc.subcore_barrier()
    pltpu.sync_copy(sp_tot.at[pl.ds((sid ^ 1) * L, L)], rb)
    pv = rb[...]
    tcp_p = jnp.sum(jnp.where(iov == 0, pv, 0.0))
    tcn_p = jnp.sum(jnp.where(iov == 1, pv, 0.0))
    g = tcp + tcp_p
    hf = h.astype(jnp.float32)
    base_a = zf + hf * tcp_p
    base_b = zf + hf * tcn_p

    def scan_body(j, carry):
        sav, sbv, acc = carry
        for k in range(2):
            o = s0 + (j * 2 + k) * L
            cpf = cnt[pl.ds(NB + o, L)].astype(jnp.float32)
            cnf = cnt[pl.ds(o, L)].astype(jnp.float32)
            sr16 = srv[pl.ds(o, L)]
            icp = plsc.cumsum(cpf)
            icn = plsc.cumsum(cnf)
            ea = sav + icp - cpf
            eb = sbv + icn - cnf
            pe = ea + cpf
            ne = eb + cnf
            d1 = g + eb
            d2 = g + ne
            j1 = jnp.where(d1 > 0, 1.0 - (g - ea) / jnp.maximum(d1, 1.0), 0.0)
            j2 = jnp.where(d2 > 0, 1.0 - (g - pe) / jnp.maximum(d2, 1.0), 0.0)
            cc = cpf + cnf
            contrib = jnp.where(cc > 0, sr16 * (j2 - j1) / jnp.maximum(cc, 1.0), 0.0)
            sav = sav + jnp.sum(cpf)
            sbv = sbv + jnp.sum(cnf)
            acc = acc + contrib
        return sav, sbv, acc

    _, _, acc = lax.fori_loop(0, HB // (L * 2), scan_body, (base_a, base_b, zf))

    wb[...] = acc
    pltpu.sync_copy(wb, sp_loss.at[pl.ds(sid * L, L)])
    plsc.subcore_barrier()
    pltpu.sync_copy(sp_loss.at[pl.ds((sid ^ 1) * L, L)], rb)
    loss = jnp.sum(acc + rb[...])

    @pl.when(h == 0)
    def _():
        wb[...] = jnp.where(iov == 0, loss, 0.0)
        pltpu.sync_copy(wb, out_hbm.at[pl.ds(sample * L, L)])


def kernel(inputs, targets):
    x1 = inputs.reshape(S * P)
    t1 = targets.astype(jnp.int32).reshape(S * P)
    out = _lovasz_sc(x1, t1)
    return jnp.mean(out.reshape(S, L)[:, 0])


# speedup vs baseline: 33.9432x; 1.0563x over previous
import functools

import jax
import jax.numpy as jnp
from jax import lax
from jax.experimental import pallas as pl
from jax.experimental.pallas import tpu as pltpu
from jax.experimental.pallas import tpu_sc as plsc

S = 16
P = 512 * 512
HALF = P // 2
NB = 16384
HB = NB // 2
CH = 8192
NCH = HALF // CH
L = 16
U = 16

_mesh = plsc.VectorSubcoreMesh(core_axis_name="c", subcore_axis_name="s")


@functools.partial(
    pl.kernel,
    out_type=jax.ShapeDtypeStruct((S * L,), jnp.float32),
    mesh=_mesh,
    compiler_params=pltpu.CompilerParams(needs_layout_passes=False),
    scratch_types=[
        pltpu.VMEM((CH,), jnp.float32),
        pltpu.VMEM((CH,), jnp.float32),
        pltpu.VMEM((CH,), jnp.int32),
        pltpu.VMEM((CH,), jnp.int32),
        pltpu.VMEM((2 * NB,), jnp.int32),
        pltpu.VMEM((NB,), jnp.float32),
        pltpu.VMEM((L,), jnp.float32),
        pltpu.VMEM((L,), jnp.float32),
        pltpu.VMEM_SHARED((16 * 2 * HB,), jnp.int32),
        pltpu.VMEM_SHARED((16 * HB,), jnp.float32),
        pltpu.VMEM_SHARED((16 * L,), jnp.float32),
        pltpu.VMEM_SHARED((16 * L,), jnp.float32),
        pltpu.VMEM_SHARED((16 * L,), jnp.float32),
        pltpu.SemaphoreType.DMA,
        pltpu.SemaphoreType.DMA,
        pltpu.SemaphoreType.DMA,
        pltpu.SemaphoreType.DMA,
    ],
)
def _lovasz_sc(x_hbm, t_hbm, out_hbm, xb0, xb1, tb0, tb1, cnt, srv, wb, rb,
               sp_cnt, sp_sr, sp_max, sp_tot, sp_loss,
               smx0, smx1, smt0, smt1):
    cid = lax.axis_index("c")
    sid = lax.axis_index("s")
    sample = cid * 8 + sid // 2
    h = sid % 2
    base = sample * P + h * HALF
    iov = lax.broadcasted_iota(jnp.int32, (L,), 0)
    zi = jnp.zeros((L,), jnp.int32)
    zf = jnp.zeros((L,), jnp.float32)
    xb = [xb0, xb1]
    tb = [tb0, tb1]
    smx = [smx0, smx1]
    smt = [smt0, smt1]

    def xsrc(i):
        return x_hbm.at[pl.ds(base + i * CH, CH)]

    def tsrc(i):
        return t_hbm.at[pl.ds(base + i * CH, CH)]

    pltpu.async_copy(xsrc(0), xb0, smx0)
    m = zf
    for i in range(NCH):
        p = i % 2
        if i + 1 < NCH:
            pltpu.async_copy(xsrc(i + 1), xb[1 - p], smx[1 - p])
        pltpu.make_async_copy(xsrc(i), xb[p], smx[p]).wait()
        buf = xb[p]

        @plsc.parallel_loop(0, CH // L, unroll=U, carry=m)
        def p1v(j, mm, buf=buf):
            return jnp.maximum(mm, jnp.abs(buf[pl.ds(j * L, L)]))

        m = p1v

    pltpu.async_copy(xsrc(0), xb0, smx0)
    pltpu.async_copy(tsrc(0), tb0, smt0)

    wb[...] = m
    pltpu.sync_copy(wb, sp_max.at[pl.ds(sid * L, L)])
    plsc.subcore_barrier()
    pltpu.sync_copy(sp_max.at[pl.ds((sid ^ 1) * L, L)], rb)
    emax = 1.0 + jnp.max(jnp.maximum(m, rb[...]))

    @plsc.parallel_loop(0, (2 * NB) // L, unroll=2 * U)
    def _zc(j):
        cnt[pl.ds(j * L, L)] = zi

    @plsc.parallel_loop(0, NB // L, unroll=2 * U)
    def _zs(j):
        srv[pl.ds(j * L, L)] = zf

    emaxv = zf + emax
    scale = NB / jnp.maximum(emaxv, 1e-30)
    c0s = (emaxv - 1.0) * scale
    bmax = jnp.full((L,), float(NB - 1), jnp.float32)
    onei = jnp.ones((L,), jnp.int32)

    for i in range(NCH):
        p = i % 2
        if i + 1 < NCH:
            pltpu.async_copy(xsrc(i + 1), xb[1 - p], smx[1 - p])
            pltpu.async_copy(tsrc(i + 1), tb[1 - p], smt[1 - p])
        pltpu.make_async_copy(xsrc(i), xb[p], smx[p]).wait()
        pltpu.make_async_copy(tsrc(i), tb[p], smt[p]).wait()
        bx, bt = xb[p], tb[p]

        @plsc.parallel_loop(0, CH // L, unroll=U)
        def p2v(j, bx=bx, bt=bt):
            o = j * L
            xx = bx[pl.ds(o, L)]
            ti = bt[pl.ds(o, L)]
            tf = ti.astype(jnp.float32)
            xs = xx * (2.0 * tf - 1.0)
            relu = jnp.maximum(1.0 - xs, 0.0)
            bf = jnp.minimum(c0s + xs * scale, bmax)
            b = bf.astype(jnp.int32)
            plsc.addupdate_scatter(cnt, [b + ti * NB], onei)
            plsc.addupdate_scatter(srv, [b], relu)

    s0 = h * HB
    ps0 = HB - s0
    pltpu.sync_copy(cnt.at[pl.ds(ps0, HB)],
                    sp_cnt.at[pl.ds(sid * 2 * HB, HB)])
    pltpu.sync_copy(cnt.at[pl.ds(NB + ps0, HB)],
                    sp_cnt.at[pl.ds(sid * 2 * HB + HB, HB)])
    pltpu.sync_copy(srv.at[pl.ds(ps0, HB)],
                    sp_sr.at[pl.ds(sid * HB, HB)])
    plsc.subcore_barrier()

    pbase_c = (sid ^ 1) * 2 * HB
    pbase_s = (sid ^ 1) * HB

    pltpu.async_copy(sp_cnt.at[pl.ds(pbase_c, HB)], tb0.at[pl.ds(0, HB)], smt0)
    pltpu.async_copy(sp_cnt.at[pl.ds(pbase_c + HB, HB)], tb1.at[pl.ds(0, HB)], smt1)
    pltpu.async_copy(sp_sr.at[pl.ds(pbase_s, HB)], xb0.at[pl.ds(0, HB)], smx0)

    pltpu.make_async_copy(sp_cnt.at[pl.ds(pbase_c, HB)], tb0.at[pl.ds(0, HB)], smt0).wait()

    @plsc.parallel_loop(0, HB // L, unroll=U, carry=zi)
    def accn(j, acc):
        o = j * L
        v = cnt[pl.ds(s0 + o, L)] + tb0[pl.ds(o, L)]
        cnt[pl.ds(s0 + o, L)] = v
        return acc + v

    tcn = jnp.sum(accn).astype(jnp.float32)

    pltpu.make_async_copy(sp_cnt.at[pl.ds(pbase_c + HB, HB)], tb1.at[pl.ds(0, HB)], smt1).wait()

    @plsc.parallel_loop(0, HB // L, unroll=U, carry=zi)
    def accp(j, acc):
        o = j * L
        v = cnt[pl.ds(NB + s0 + o, L)] + tb1[pl.ds(o, L)]
        cnt[pl.ds(NB + s0 + o, L)] = v
        return acc + v

    tcp = jnp.sum(accp).astype(jnp.float32)

    pltpu.make_async_copy(sp_sr.at[pl.ds(pbase_s, HB)], xb0.at[pl.ds(0, HB)], smx0).wait()

    @plsc.parallel_loop(0, HB // L, unroll=U)
    def _addsr(j):
        o = j * L
        srv[pl.ds(s0 + o, L)] = srv[pl.ds(s0 + o, L)] + xb0[pl.ds(o, L)]

    wb[...] = jnp.where(iov == 0, tcp, jnp.where(iov == 1, tcn, 0.0))
    pltpu.sync_copy(wb, sp_tot.at[pl.ds(sid * L, L)])
    plsc.subcore_barrier()
    pltpu.sync_copy(sp_tot.at[pl.ds((sid ^ 1) * L, L)], rb)
    pv = rb[...]
    tcp_p = jnp.sum(jnp.where(iov == 0, pv, 0.0))
    tcn_p = jnp.sum(jnp.where(iov == 1, pv, 0.0))
    g = tcp + tcp_p
    hf = h.astype(jnp.float32)
    base_a = zf + hf * tcp_p
    base_b = zf + hf * tcn_p

    def scan_body(j, carry):
        sav, sbv, acc = carry
        for k in range(2):
            o = s0 + (j * 2 + k) * L
            cpf = cnt[pl.ds(NB + o, L)].astype(jnp.float32)
            cnf = cnt[pl.ds(o, L)].astype(jnp.float32)
            sr16 = srv[pl.ds(o, L)]
            icp = plsc.cumsum(cpf)
            icn = plsc.cumsum(cnf)
            ea = sav + icp - cpf
            eb = sbv + icn - cnf
            pe = ea + cpf
            ne = eb + cnf
            d1 = g + eb
            d2 = g + ne
            j1 = jnp.where(d1 > 0, 1.0 - (g - ea) / jnp.maximum(d1, 1.0), 0.0)
            j2 = jnp.where(d2 > 0, 1.0 - (g - pe) / jnp.maximum(d2, 1.0), 0.0)
            cc = cpf + cnf
            contrib = jnp.where(cc > 0, sr16 * (j2 - j1) / jnp.maximum(cc, 1.0), 0.0)
            sav = sav + jnp.sum(cpf)
            sbv = sbv + jnp.sum(cnf)
            acc = acc + contrib
        return sav, sbv, acc

    _, _, acc = lax.fori_loop(0, HB // (L * 2), scan_body, (base_a, base_b, zf))

    wb[...] = acc
    pltpu.sync_copy(wb, sp_loss.at[pl.ds(sid * L, L)])
    plsc.subcore_barrier()
    pltpu.sync_copy(sp_loss.at[pl.ds((sid ^ 1) * L, L)], rb)
    loss = jnp.sum(acc + rb[...])

    @pl.when(h == 0)
    def _():
        wb[...] = jnp.where(iov == 0, loss, 0.0)
        pltpu.sync_copy(wb, out_hbm.at[pl.ds(sample * L, L)])


def kernel(inputs, targets):
    x1 = inputs.reshape(S * P)
    t1 = targets.astype(jnp.int32).reshape(S * P)
    out = _lovasz_sc(x1, t1)
    return jnp.mean(out.reshape(S, L)[:, 0])
